# baseline (speedup 1.0000x reference)
import functools

import jax
import jax.numpy as jnp
from jax.experimental import pallas as pl
from jax.experimental.pallas import tpu as pltpu

_EPS = 1e-5


def _conv3_kernel(x_ref, w_ref, y_ref, st_ref, *, B, H, W):
    HW = H * W
    LB = B * HW
    xb = jnp.concatenate([x_ref[b] for b in range(B)],
                         axis=1).astype(jnp.bfloat16)
    lane = jax.lax.broadcasted_iota(jnp.int32, (1, LB), 1)
    wpos = lane % W
    hpos = (lane // W) % H
    taps = []
    for dy in range(3):
        for dx in range(3):
            off = (dy - 1) * W + (dx - 1)
            if off == 0:
                sheet = xb
            else:
                r = off % LB
                sheet = jnp.concatenate([xb[:, r:], xb[:, :r]], axis=1)
            conds = []
            if dy == 0:
                conds.append(hpos >= 1)
            if dy == 2:
                conds.append(hpos <= H - 2)
            if dx == 0:
                conds.append(wpos >= 1)
            if dx == 2:
                conds.append(wpos <= W - 2)
            if conds:
                m = conds[0]
                for c in conds[1:]:
                    m = jnp.logical_and(m, c)
                sheet = jnp.where(m, sheet, jnp.bfloat16(0))
            taps.append(sheet)
    a9 = jnp.concatenate(taps, axis=0)
    y = jax.lax.dot_general(w_ref[...], a9, (((1,), (0,)), ((), ())),
                            preferred_element_type=jnp.float32)
    s = jnp.sum(y, axis=1, keepdims=True)
    ss = jnp.sum(y * y, axis=1, keepdims=True)
    st_ref[0] = jnp.concatenate([s, ss], axis=1)
    y_ref[0] = y.astype(jnp.bfloat16)


def _bn_conv1_kernel(y_ref, sc_ref, sh_ref, w_ref, z_ref, st_ref):
    a = jnp.maximum(y_ref[0].astype(jnp.float32) * sc_ref[...] + sh_ref[...],
                    0.0)
    z = jax.lax.dot_general(w_ref[...], a.astype(jnp.bfloat16),
                            (((1,), (0,)), ((), ())),
                            preferred_element_type=jnp.float32)
    s = jnp.sum(z, axis=1, keepdims=True)
    ss = jnp.sum(z * z, axis=1, keepdims=True)
    st_ref[0] = jnp.concatenate([s, ss], axis=1)
    z_ref[0] = z.astype(jnp.bfloat16)


def _bn_out_kernel(z_ref, sc_ref, sh_ref, o_ref, *, B, HW):
    o = jnp.maximum(z_ref[0].astype(jnp.float32) * sc_ref[...] + sh_ref[...],
                    0.0)
    for b in range(B):
        o_ref[b] = o[:, b * HW:(b + 1) * HW]


def _fold_bn(st, gamma, beta, count):
    tot = jnp.sum(st.astype(jnp.float32), axis=0)
    mean = tot[:, 0] / count
    var = tot[:, 1] / count - mean * mean
    scale = gamma * jax.lax.rsqrt(var + _EPS)
    shift = beta - mean * scale
    return scale.reshape(-1, 1), shift.reshape(-1, 1)


@jax.jit
def _forward(x_nchw, w3_hwio, w1, gamma1, beta1, gamma2, beta2):
    N, Cin, H, W = x_nchw.shape
    Cout = w3_hwio.shape[-1]
    HW = H * W
    B = 32 if N % 32 == 0 else (8 if N % 8 == 0 else 1)
    S = N // B
    LB = B * HW
    parallel = pltpu.CompilerParams(dimension_semantics=("arbitrary",))

    x3 = x_nchw.reshape(N, Cin, HW)
    w9t = jnp.transpose(w3_hwio, (3, 0, 1, 2)).reshape(
        Cout, 9 * Cin).astype(jnp.bfloat16)
    w1t = jnp.transpose(w1).astype(jnp.bfloat16)

    y1, st1 = pl.pallas_call(
        functools.partial(_conv3_kernel, B=B, H=H, W=W),
        grid=(S,),
        in_specs=[
            pl.BlockSpec((B, Cin, HW), lambda i: (i, 0, 0)),
            pl.BlockSpec((Cout, 9 * Cin), lambda i: (0, 0)),
        ],
        out_specs=[
            pl.BlockSpec((1, Cout, LB), lambda i: (i, 0, 0)),
            pl.BlockSpec((1, Cout, 2), lambda i: (i, 0, 0)),
        ],
        out_shape=[
            jax.ShapeDtypeStruct((S, Cout, LB), jnp.bfloat16),
            jax.ShapeDtypeStruct((S, Cout, 2), jnp.float32),
        ],
        compiler_params=parallel,
    )(x3, w9t)

    sc1, sh1 = _fold_bn(st1, gamma1, beta1, N * HW)

    z, st2 = pl.pallas_call(
        _bn_conv1_kernel,
        grid=(S,),
        in_specs=[
            pl.BlockSpec((1, Cout, LB), lambda i: (i, 0, 0)),
            pl.BlockSpec((Cout, 1), lambda i: (0, 0)),
            pl.BlockSpec((Cout, 1), lambda i: (0, 0)),
            pl.BlockSpec((Cout, Cout), lambda i: (0, 0)),
        ],
        out_specs=[
            pl.BlockSpec((1, Cout, LB), lambda i: (i, 0, 0)),
            pl.BlockSpec((1, Cout, 2), lambda i: (i, 0, 0)),
        ],
        out_shape=[
            jax.ShapeDtypeStruct((S, Cout, LB), jnp.bfloat16),
            jax.ShapeDtypeStruct((S, Cout, 2), jnp.float32),
        ],
        compiler_params=parallel,
    )(y1, sc1, sh1, w1t)

    sc2, sh2 = _fold_bn(st2, gamma2, beta2, N * HW)

    out3 = pl.pallas_call(
        functools.partial(_bn_out_kernel, B=B, HW=HW),
        grid=(S,),
        in_specs=[
            pl.BlockSpec((1, Cout, LB), lambda i: (i, 0, 0)),
            pl.BlockSpec((Cout, 1), lambda i: (0, 0)),
            pl.BlockSpec((Cout, 1), lambda i: (0, 0)),
        ],
        out_specs=pl.BlockSpec((B, Cout, HW), lambda i: (i, 0, 0)),
        out_shape=jax.ShapeDtypeStruct((N, Cout, HW), jnp.float32),
        compiler_params=parallel,
    )(z, sc2, sh2)

    return out3.reshape(N, Cout, H, W)


def kernel(x_nchw, w3_hwio, w1, gamma1, beta1, gamma2, beta2):
    return _forward(x_nchw, w3_hwio, w1, gamma1, beta1, gamma2, beta2)


# speedup vs baseline: 1.0005x; 1.0005x over previous
import functools

import jax
import jax.numpy as jnp
from jax.experimental import pallas as pl
from jax.experimental.pallas import tpu as pltpu

_EPS = 1e-5


def _conv3_kernel(x_ref, w_ref, y_ref, st_ref, *, B, H, W):
    HW = H * W
    LB = B * HW
    xb = jnp.concatenate([x_ref[b] for b in range(B)], axis=1)
    lane = jax.lax.broadcasted_iota(jnp.int32, (1, LB), 1)
    wpos = lane % W
    hpos = (lane // W) % H
    taps = []
    for dy in range(3):
        for dx in range(3):
            off = (dy - 1) * W + (dx - 1)
            if off == 0:
                sheet = xb
            else:
                r = off % LB
                sheet = jnp.concatenate([xb[:, r:], xb[:, :r]], axis=1)
            conds = []
            if dy == 0:
                conds.append(hpos >= 1)
            if dy == 2:
                conds.append(hpos <= H - 2)
            if dx == 0:
                conds.append(wpos >= 1)
            if dx == 2:
                conds.append(wpos <= W - 2)
            if conds:
                m = conds[0]
                for c in conds[1:]:
                    m = jnp.logical_and(m, c)
                sheet = jnp.where(m, sheet, jnp.bfloat16(0))
            taps.append(sheet)
    a9 = jnp.concatenate(taps, axis=0)
    y = jax.lax.dot_general(w_ref[...], a9, (((1,), (0,)), ((), ())),
                            preferred_element_type=jnp.float32)
    s = jnp.sum(y, axis=1, keepdims=True)
    ss = jnp.sum(y * y, axis=1, keepdims=True)
    st_ref[0] = jnp.concatenate([s, ss], axis=1)
    y_ref[0] = y.astype(jnp.bfloat16)


def _bn_conv1_kernel(y_ref, sc_ref, sh_ref, w_ref, z_ref, st_ref):
    a = jnp.maximum(y_ref[0].astype(jnp.float32) * sc_ref[...] + sh_ref[...],
                    0.0)
    z = jax.lax.dot_general(w_ref[...], a.astype(jnp.bfloat16),
                            (((1,), (0,)), ((), ())),
                            preferred_element_type=jnp.float32)
    s = jnp.sum(z, axis=1, keepdims=True)
    ss = jnp.sum(z * z, axis=1, keepdims=True)
    st_ref[0] = jnp.concatenate([s, ss], axis=1)
    z_ref[0] = z.astype(jnp.bfloat16)


def _bn_out_kernel(z_ref, sc_ref, sh_ref, o_ref, *, B, HW):
    o = jnp.maximum(z_ref[0].astype(jnp.float32) * sc_ref[...] + sh_ref[...],
                    0.0).astype(jnp.bfloat16)
    for b in range(B):
        o_ref[b] = o[:, b * HW:(b + 1) * HW]


def _fold_bn(st, gamma, beta, count):
    tot = jnp.sum(st.astype(jnp.float32), axis=0)
    mean = tot[:, 0] / count
    var = tot[:, 1] / count - mean * mean
    scale = gamma * jax.lax.rsqrt(var + _EPS)
    shift = beta - mean * scale
    return scale.reshape(-1, 1), shift.reshape(-1, 1)


@jax.jit
def _forward(x_nchw, w3_hwio, w1, gamma1, beta1, gamma2, beta2):
    N, Cin, H, W = x_nchw.shape
    Cout = w3_hwio.shape[-1]
    HW = H * W
    B = 32 if N % 32 == 0 else (8 if N % 8 == 0 else 1)
    S = N // B
    LB = B * HW
    parallel = pltpu.CompilerParams(dimension_semantics=("parallel",))

    x3 = x_nchw.reshape(N, Cin, HW).astype(jnp.bfloat16)
    w9t = jnp.transpose(w3_hwio, (3, 0, 1, 2)).reshape(
        Cout, 9 * Cin).astype(jnp.bfloat16)
    w1t = jnp.transpose(w1).astype(jnp.bfloat16)

    y1, st1 = pl.pallas_call(
        functools.partial(_conv3_kernel, B=B, H=H, W=W),
        grid=(S,),
        in_specs=[
            pl.BlockSpec((B, Cin, HW), lambda i: (i, 0, 0)),
            pl.BlockSpec((Cout, 9 * Cin), lambda i: (0, 0)),
        ],
        out_specs=[
            pl.BlockSpec((1, Cout, LB), lambda i: (i, 0, 0)),
            pl.BlockSpec((1, Cout, 2), lambda i: (i, 0, 0)),
        ],
        out_shape=[
            jax.ShapeDtypeStruct((S, Cout, LB), jnp.bfloat16),
            jax.ShapeDtypeStruct((S, Cout, 2), jnp.float32),
        ],
        compiler_params=parallel,
    )(x3, w9t)

    sc1, sh1 = _fold_bn(st1, gamma1, beta1, N * HW)

    z, st2 = pl.pallas_call(
        _bn_conv1_kernel,
        grid=(S,),
        in_specs=[
            pl.BlockSpec((1, Cout, LB), lambda i: (i, 0, 0)),
            pl.BlockSpec((Cout, 1), lambda i: (0, 0)),
            pl.BlockSpec((Cout, 1), lambda i: (0, 0)),
            pl.BlockSpec((Cout, Cout), lambda i: (0, 0)),
        ],
        out_specs=[
            pl.BlockSpec((1, Cout, LB), lambda i: (i, 0, 0)),
            pl.BlockSpec((1, Cout, 2), lambda i: (i, 0, 0)),
        ],
        out_shape=[
            jax.ShapeDtypeStruct((S, Cout, LB), jnp.bfloat16),
            jax.ShapeDtypeStruct((S, Cout, 2), jnp.float32),
        ],
        compiler_params=parallel,
    )(y1, sc1, sh1, w1t)

    sc2, sh2 = _fold_bn(st2, gamma2, beta2, N * HW)

    out3 = pl.pallas_call(
        functools.partial(_bn_out_kernel, B=B, HW=HW),
        grid=(S,),
        in_specs=[
            pl.BlockSpec((1, Cout, LB), lambda i: (i, 0, 0)),
            pl.BlockSpec((Cout, 1), lambda i: (0, 0)),
            pl.BlockSpec((Cout, 1), lambda i: (0, 0)),
        ],
        out_specs=pl.BlockSpec((B, Cout, HW), lambda i: (i, 0, 0)),
        out_shape=jax.ShapeDtypeStruct((N, Cout, HW), jnp.bfloat16),
        compiler_params=parallel,
    )(z, sc2, sh2)

    return out3.reshape(N, Cout, H, W).astype(jnp.float32)


def kernel(x_nchw, w3_hwio, w1, gamma1, beta1, gamma2, beta2):
    return _forward(x_nchw, w3_hwio, w1, gamma1, beta1, gamma2, beta2)


# speedup vs baseline: 1.1346x; 1.1340x over previous
import functools

import jax
import jax.numpy as jnp
from jax.experimental import pallas as pl
from jax.experimental.pallas import tpu as pltpu

_EPS = 1e-5


def _conv3_kernel(x_ref, w_ref, y_ref, st_ref, *, B, H, W, Cout):
    HW = H * W
    LB = B * HW
    xb = jnp.concatenate([x_ref[b] for b in range(B)], axis=1)
    lane = jax.lax.broadcasted_iota(jnp.int32, (1, LB), 1)
    wpos = lane % W
    hpos = (lane // W) % H
    left = jnp.where(wpos >= 1,
                     jnp.concatenate([xb[:, LB - 1:], xb[:, :LB - 1]], axis=1),
                     jnp.bfloat16(0))
    right = jnp.where(wpos <= W - 2,
                      jnp.concatenate([xb[:, 1:], xb[:, :1]], axis=1),
                      jnp.bfloat16(0))
    a3 = jnp.concatenate([left, xb, right], axis=0)
    c = jax.lax.dot_general(w_ref[...], a3, (((1,), (0,)), ((), ())),
                            preferred_element_type=jnp.float32)
    c0 = c[:Cout]
    c2 = c[2 * Cout:]
    up = jnp.where(hpos >= 1,
                   jnp.concatenate([c0[:, LB - W:], c0[:, :LB - W]], axis=1),
                   0.0)
    dn = jnp.where(hpos <= H - 2,
                   jnp.concatenate([c2[:, W:], c2[:, :W]], axis=1),
                   0.0)
    y = c[Cout:2 * Cout] + up + dn
    s = jnp.sum(y, axis=1, keepdims=True)
    ss = jnp.sum(y * y, axis=1, keepdims=True)
    st_ref[0] = jnp.concatenate([s, ss], axis=1)
    y_ref[0] = y.astype(jnp.bfloat16)


def _bn_conv1_kernel(y_ref, sc_ref, sh_ref, w_ref, z_ref, st_ref):
    a = jnp.maximum(y_ref[0].astype(jnp.float32) * sc_ref[...] + sh_ref[...],
                    0.0)
    z = jax.lax.dot_general(w_ref[...], a.astype(jnp.bfloat16),
                            (((1,), (0,)), ((), ())),
                            preferred_element_type=jnp.float32)
    s = jnp.sum(z, axis=1, keepdims=True)
    ss = jnp.sum(z * z, axis=1, keepdims=True)
    st_ref[0] = jnp.concatenate([s, ss], axis=1)
    z_ref[0] = z.astype(jnp.bfloat16)


def _bn_out_kernel(z_ref, sc_ref, sh_ref, o_ref, *, B, HW):
    o = jnp.maximum(z_ref[0].astype(jnp.float32) * sc_ref[...] + sh_ref[...],
                    0.0).astype(jnp.bfloat16)
    for b in range(B):
        o_ref[b] = o[:, b * HW:(b + 1) * HW]


def _fold_bn(st, gamma, beta, count):
    tot = jnp.sum(st.astype(jnp.float32), axis=0)
    mean = tot[:, 0] / count
    var = tot[:, 1] / count - mean * mean
    scale = gamma * jax.lax.rsqrt(var + _EPS)
    shift = beta - mean * scale
    return scale.reshape(-1, 1), shift.reshape(-1, 1)


@jax.jit
def _forward(x_nchw, w3_hwio, w1, gamma1, beta1, gamma2, beta2):
    N, Cin, H, W = x_nchw.shape
    Cout = w3_hwio.shape[-1]
    HW = H * W
    B = 64 if N % 64 == 0 else (8 if N % 8 == 0 else 1)
    S = N // B
    LB = B * HW
    parallel = pltpu.CompilerParams(dimension_semantics=("parallel",))

    x3 = x_nchw.reshape(N, Cin, HW).astype(jnp.bfloat16)
    wc = jnp.transpose(w3_hwio, (0, 3, 1, 2)).reshape(
        3 * Cout, 3 * Cin).astype(jnp.bfloat16)
    w1t = jnp.transpose(w1).astype(jnp.bfloat16)

    y1, st1 = pl.pallas_call(
        functools.partial(_conv3_kernel, B=B, H=H, W=W, Cout=Cout),
        grid=(S,),
        in_specs=[
            pl.BlockSpec((B, Cin, HW), lambda i: (i, 0, 0)),
            pl.BlockSpec((3 * Cout, 3 * Cin), lambda i: (0, 0)),
        ],
        out_specs=[
            pl.BlockSpec((1, Cout, LB), lambda i: (i, 0, 0)),
            pl.BlockSpec((1, Cout, 2), lambda i: (i, 0, 0)),
        ],
        out_shape=[
            jax.ShapeDtypeStruct((S, Cout, LB), jnp.bfloat16),
            jax.ShapeDtypeStruct((S, Cout, 2), jnp.float32),
        ],
        compiler_params=parallel,
    )(x3, wc)

    sc1, sh1 = _fold_bn(st1, gamma1, beta1, N * HW)

    z, st2 = pl.pallas_call(
        _bn_conv1_kernel,
        grid=(S,),
        in_specs=[
            pl.BlockSpec((1, Cout, LB), lambda i: (i, 0, 0)),
            pl.BlockSpec((Cout, 1), lambda i: (0, 0)),
            pl.BlockSpec((Cout, 1), lambda i: (0, 0)),
            pl.BlockSpec((Cout, Cout), lambda i: (0, 0)),
        ],
        out_specs=[
            pl.BlockSpec((1, Cout, LB), lambda i: (i, 0, 0)),
            pl.BlockSpec((1, Cout, 2), lambda i: (i, 0, 0)),
        ],
        out_shape=[
            jax.ShapeDtypeStruct((S, Cout, LB), jnp.bfloat16),
            jax.ShapeDtypeStruct((S, Cout, 2), jnp.float32),
        ],
        compiler_params=parallel,
    )(y1, sc1, sh1, w1t)

    sc2, sh2 = _fold_bn(st2, gamma2, beta2, N * HW)

    out3 = pl.pallas_call(
        functools.partial(_bn_out_kernel, B=B, HW=HW),
        grid=(S,),
        in_specs=[
            pl.BlockSpec((1, Cout, LB), lambda i: (i, 0, 0)),
            pl.BlockSpec((Cout, 1), lambda i: (0, 0)),
            pl.BlockSpec((Cout, 1), lambda i: (0, 0)),
        ],
        out_specs=pl.BlockSpec((B, Cout, HW), lambda i: (i, 0, 0)),
        out_shape=jax.ShapeDtypeStruct((N, Cout, HW), jnp.bfloat16),
        compiler_params=parallel,
    )(z, sc2, sh2)

    return out3.reshape(N, Cout, H, W).astype(jnp.float32)


def kernel(x_nchw, w3_hwio, w1, gamma1, beta1, gamma2, beta2):
    return _forward(x_nchw, w3_hwio, w1, gamma1, beta1, gamma2, beta2)


# speedup vs baseline: 1.1535x; 1.0166x over previous
import functools

import jax
import jax.numpy as jnp
from jax.experimental import pallas as pl
from jax.experimental.pallas import tpu as pltpu

_EPS = 1e-5


def _conv3_kernel(x_ref, w_ref, y_ref, st_ref, *, B, H, W, Cout):
    HW = H * W
    LB = B * HW
    xb = jnp.concatenate([x_ref[b] for b in range(B)], axis=1)
    lane = jax.lax.broadcasted_iota(jnp.int32, (1, LB), 1)
    wpos = lane % W
    hpos = (lane // W) % H
    left = jnp.where(wpos >= 1,
                     jnp.concatenate([xb[:, LB - 1:], xb[:, :LB - 1]], axis=1),
                     jnp.bfloat16(0))
    right = jnp.where(wpos <= W - 2,
                      jnp.concatenate([xb[:, 1:], xb[:, :1]], axis=1),
                      jnp.bfloat16(0))
    a3 = jnp.concatenate([left, xb, right], axis=0)
    c = jax.lax.dot_general(w_ref[...], a3, (((1,), (0,)), ((), ())),
                            preferred_element_type=jnp.float32)
    c0 = c[:Cout]
    c2 = c[2 * Cout:]
    up = jnp.where(hpos >= 1,
                   jnp.concatenate([c0[:, LB - W:], c0[:, :LB - W]], axis=1),
                   0.0)
    dn = jnp.where(hpos <= H - 2,
                   jnp.concatenate([c2[:, W:], c2[:, :W]], axis=1),
                   0.0)
    y = c[Cout:2 * Cout] + up + dn
    s = jnp.sum(y, axis=1, keepdims=True)
    ss = jnp.sum(y * y, axis=1, keepdims=True)
    st_ref[0] = jnp.concatenate([s, ss], axis=1)
    y_ref[0] = y.astype(jnp.bfloat16)


def _bn_conv_bn_kernel(y_ref, sc1_ref, sh1_ref, w_ref, g2_ref, b2_ref,
                       o_ref, acc_ref, sc2_ref, sh2_ref, *, B, HW, count):
    ph = pl.program_id(0)
    i = pl.program_id(1)

    @pl.when(jnp.logical_and(ph == 0, i == 0))
    def _init():
        acc_ref[...] = jnp.zeros_like(acc_ref)

    a = jnp.maximum(
        y_ref[0].astype(jnp.float32) * sc1_ref[...] + sh1_ref[...], 0.0)
    z = jax.lax.dot_general(w_ref[...], a.astype(jnp.bfloat16),
                            (((1,), (0,)), ((), ())),
                            preferred_element_type=jnp.float32)

    @pl.when(ph == 0)
    def _stats():
        s = jnp.sum(z, axis=1, keepdims=True)
        ss = jnp.sum(z * z, axis=1, keepdims=True)
        acc_ref[...] += jnp.concatenate([s, ss], axis=1)

    @pl.when(jnp.logical_and(ph == 1, i == 0))
    def _fold():
        tot = acc_ref[...]
        mean = tot[:, 0:1] / count
        var = tot[:, 1:2] / count - mean * mean
        sc2 = g2_ref[...] * jax.lax.rsqrt(var + _EPS)
        sc2_ref[...] = sc2
        sh2_ref[...] = b2_ref[...] - mean * sc2

    @pl.when(ph == 1)
    def _emit():
        o = jnp.maximum(z * sc2_ref[...] + sh2_ref[...],
                        0.0).astype(jnp.bfloat16)
        for b in range(B):
            o_ref[b] = o[:, b * HW:(b + 1) * HW]


def _fold_bn(st, gamma, beta, count):
    tot = jnp.sum(st.astype(jnp.float32), axis=0)
    mean = tot[:, 0] / count
    var = tot[:, 1] / count - mean * mean
    scale = gamma * jax.lax.rsqrt(var + _EPS)
    shift = beta - mean * scale
    return scale.reshape(-1, 1), shift.reshape(-1, 1)


@jax.jit
def _forward(x_nchw, w3_hwio, w1, gamma1, beta1, gamma2, beta2):
    N, Cin, H, W = x_nchw.shape
    Cout = w3_hwio.shape[-1]
    HW = H * W
    B = 64 if N % 64 == 0 else (8 if N % 8 == 0 else 1)
    S = N // B
    LB = B * HW
    parallel = pltpu.CompilerParams(dimension_semantics=("parallel",))

    x3 = x_nchw.reshape(N, Cin, HW).astype(jnp.bfloat16)
    wc = jnp.transpose(w3_hwio, (0, 3, 1, 2)).reshape(
        3 * Cout, 3 * Cin).astype(jnp.bfloat16)
    w1t = jnp.transpose(w1).astype(jnp.bfloat16)

    y1, st1 = pl.pallas_call(
        functools.partial(_conv3_kernel, B=B, H=H, W=W, Cout=Cout),
        grid=(S,),
        in_specs=[
            pl.BlockSpec((B, Cin, HW), lambda i: (i, 0, 0)),
            pl.BlockSpec((3 * Cout, 3 * Cin), lambda i: (0, 0)),
        ],
        out_specs=[
            pl.BlockSpec((1, Cout, LB), lambda i: (i, 0, 0)),
            pl.BlockSpec((1, Cout, 2), lambda i: (i, 0, 0)),
        ],
        out_shape=[
            jax.ShapeDtypeStruct((S, Cout, LB), jnp.bfloat16),
            jax.ShapeDtypeStruct((S, Cout, 2), jnp.float32),
        ],
        compiler_params=parallel,
    )(x3, wc)

    sc1, sh1 = _fold_bn(st1, gamma1, beta1, N * HW)

    out3 = pl.pallas_call(
        functools.partial(_bn_conv_bn_kernel, B=B, HW=HW, count=N * HW),
        grid=(2, S),
        in_specs=[
            pl.BlockSpec((1, Cout, LB), lambda p, i: (i, 0, 0)),
            pl.BlockSpec((Cout, 1), lambda p, i: (0, 0)),
            pl.BlockSpec((Cout, 1), lambda p, i: (0, 0)),
            pl.BlockSpec((Cout, Cout), lambda p, i: (0, 0)),
            pl.BlockSpec((Cout, 1), lambda p, i: (0, 0)),
            pl.BlockSpec((Cout, 1), lambda p, i: (0, 0)),
        ],
        out_specs=pl.BlockSpec((B, Cout, HW), lambda p, i: (i * p, 0, 0)),
        out_shape=jax.ShapeDtypeStruct((N, Cout, HW), jnp.bfloat16),
        scratch_shapes=[
            pltpu.VMEM((Cout, 2), jnp.float32),
            pltpu.VMEM((Cout, 1), jnp.float32),
            pltpu.VMEM((Cout, 1), jnp.float32),
        ],
        compiler_params=pltpu.CompilerParams(
            dimension_semantics=("arbitrary", "arbitrary")),
    )(y1, sc1, sh1, w1t, gamma2.reshape(-1, 1), beta2.reshape(-1, 1))

    return out3.reshape(N, Cout, H, W).astype(jnp.float32)


def kernel(x_nchw, w3_hwio, w1, gamma1, beta1, gamma2, beta2):
    return _forward(x_nchw, w3_hwio, w1, gamma1, beta1, gamma2, beta2)


# speedup vs baseline: 1.2326x; 1.0686x over previous
import functools

import jax
import jax.numpy as jnp
from jax.experimental import pallas as pl
from jax.experimental.pallas import tpu as pltpu

_EPS = 1e-5


def _conv3_kernel(x_ref, w_ref, y_ref, st_ref, *, B, H, W, Cout):
    HW = H * W
    LB = B * HW
    xb = jnp.concatenate([x_ref[b] for b in range(B)],
                         axis=1).astype(jnp.bfloat16)
    lane = jax.lax.broadcasted_iota(jnp.int32, (1, LB), 1)
    wpos = lane % W
    hpos = (lane // W) % H
    left = jnp.where(wpos >= 1,
                     jnp.concatenate([xb[:, LB - 1:], xb[:, :LB - 1]], axis=1),
                     jnp.bfloat16(0))
    right = jnp.where(wpos <= W - 2,
                      jnp.concatenate([xb[:, 1:], xb[:, :1]], axis=1),
                      jnp.bfloat16(0))
    a3 = jnp.concatenate([left, xb, right], axis=0)
    c = jax.lax.dot_general(w_ref[...], a3, (((1,), (0,)), ((), ())),
                            preferred_element_type=jnp.float32)
    c0 = c[:Cout]
    c2 = c[2 * Cout:]
    up = jnp.where(hpos >= 1,
                   jnp.concatenate([c0[:, LB - W:], c0[:, :LB - W]], axis=1),
                   0.0)
    dn = jnp.where(hpos <= H - 2,
                   jnp.concatenate([c2[:, W:], c2[:, :W]], axis=1),
                   0.0)
    y = c[Cout:2 * Cout] + up + dn
    s = jnp.sum(y, axis=1, keepdims=True)
    ss = jnp.sum(y * y, axis=1, keepdims=True)
    st_ref[0] = jnp.concatenate([s, ss], axis=1)
    y_ref[0] = y.astype(jnp.bfloat16)


def _bn_conv_bn_kernel(y_ref, sc1_ref, sh1_ref, w_ref, g2_ref, b2_ref,
                       o_ref, acc_ref, sc2_ref, sh2_ref, *, B, HW, count):
    ph = pl.program_id(0)
    i = pl.program_id(1)

    @pl.when(jnp.logical_and(ph == 0, i == 0))
    def _init():
        acc_ref[...] = jnp.zeros_like(acc_ref)

    a = jnp.maximum(
        y_ref[i].astype(jnp.float32) * sc1_ref[...] + sh1_ref[...], 0.0)
    z = jax.lax.dot_general(w_ref[...], a.astype(jnp.bfloat16),
                            (((1,), (0,)), ((), ())),
                            preferred_element_type=jnp.float32)

    @pl.when(ph == 0)
    def _stats():
        s = jnp.sum(z, axis=1, keepdims=True)
        ss = jnp.sum(z * z, axis=1, keepdims=True)
        acc_ref[...] += jnp.concatenate([s, ss], axis=1)

    @pl.when(jnp.logical_and(ph == 1, i == 0))
    def _fold():
        tot = acc_ref[...]
        mean = tot[:, 0:1] / count
        var = tot[:, 1:2] / count - mean * mean
        sc2 = g2_ref[...] * jax.lax.rsqrt(var + _EPS)
        sc2_ref[...] = sc2
        sh2_ref[...] = b2_ref[...] - mean * sc2

    @pl.when(ph == 1)
    def _emit():
        o = jnp.maximum(z * sc2_ref[...] + sh2_ref[...],
                        0.0).astype(jnp.bfloat16)
        for b in range(B):
            o_ref[b] = o[:, b * HW:(b + 1) * HW]


def _fold_bn(st, gamma, beta, count):
    tot = jnp.sum(st.astype(jnp.float32), axis=0)
    mean = tot[:, 0] / count
    var = tot[:, 1] / count - mean * mean
    scale = gamma * jax.lax.rsqrt(var + _EPS)
    shift = beta - mean * scale
    return scale.reshape(-1, 1), shift.reshape(-1, 1)


@jax.jit
def _forward(x_nchw, w3_hwio, w1, gamma1, beta1, gamma2, beta2):
    N, Cin, H, W = x_nchw.shape
    Cout = w3_hwio.shape[-1]
    HW = H * W
    B = 64 if N % 64 == 0 else (8 if N % 8 == 0 else 1)
    S = N // B
    LB = B * HW
    parallel = pltpu.CompilerParams(dimension_semantics=("parallel",))

    x3 = x_nchw.reshape(N, Cin, HW)
    wc = jnp.transpose(w3_hwio, (0, 3, 1, 2)).reshape(
        3 * Cout, 3 * Cin).astype(jnp.bfloat16)
    w1t = jnp.transpose(w1).astype(jnp.bfloat16)

    y1, st1 = pl.pallas_call(
        functools.partial(_conv3_kernel, B=B, H=H, W=W, Cout=Cout),
        grid=(S,),
        in_specs=[
            pl.BlockSpec((B, Cin, HW), lambda i: (i, 0, 0)),
            pl.BlockSpec((3 * Cout, 3 * Cin), lambda i: (0, 0)),
        ],
        out_specs=[
            pl.BlockSpec((1, Cout, LB), lambda i: (i, 0, 0)),
            pl.BlockSpec((1, Cout, 2), lambda i: (i, 0, 0)),
        ],
        out_shape=[
            jax.ShapeDtypeStruct((S, Cout, LB), jnp.bfloat16),
            jax.ShapeDtypeStruct((S, Cout, 2), jnp.float32),
        ],
        compiler_params=parallel,
    )(x3, wc)

    sc1, sh1 = _fold_bn(st1, gamma1, beta1, N * HW)

    out3 = pl.pallas_call(
        functools.partial(_bn_conv_bn_kernel, B=B, HW=HW, count=N * HW),
        grid=(2, S),
        in_specs=[
            pl.BlockSpec((S, Cout, LB), lambda p, i: (0, 0, 0)),
            pl.BlockSpec((Cout, 1), lambda p, i: (0, 0)),
            pl.BlockSpec((Cout, 1), lambda p, i: (0, 0)),
            pl.BlockSpec((Cout, Cout), lambda p, i: (0, 0)),
            pl.BlockSpec((Cout, 1), lambda p, i: (0, 0)),
            pl.BlockSpec((Cout, 1), lambda p, i: (0, 0)),
        ],
        out_specs=pl.BlockSpec((B, Cout, HW), lambda p, i: (i * p, 0, 0)),
        out_shape=jax.ShapeDtypeStruct((N, Cout, HW), jnp.bfloat16),
        scratch_shapes=[
            pltpu.VMEM((Cout, 2), jnp.float32),
            pltpu.VMEM((Cout, 1), jnp.float32),
            pltpu.VMEM((Cout, 1), jnp.float32),
        ],
        compiler_params=pltpu.CompilerParams(
            dimension_semantics=("arbitrary", "arbitrary")),
    )(y1, sc1, sh1, w1t, gamma2.reshape(-1, 1), beta2.reshape(-1, 1))

    return out3.reshape(N, Cout, H, W).astype(jnp.float32)


def kernel(x_nchw, w3_hwio, w1, gamma1, beta1, gamma2, beta2):
    return _forward(x_nchw, w3_hwio, w1, gamma1, beta1, gamma2, beta2)
